# baseline (speedup 1.0000x reference)
import functools

import jax
import jax.numpy as jnp
import numpy as np
from jax import lax
from jax.experimental import pallas as pl
from jax.experimental.pallas import tpu as pltpu
from jax.experimental.pallas import tpu_sc as plsc

_B = 16384
_F = 2 * _B
_D = 32
_NC, _NS = 2, 16
_NW = _NC * _NS
_PER_W = _F // _NW
_CHUNK = 128
_NCHUNK = _PER_W // _CHUNK
_BBLK = 16


def _eps():
  nk = jax.random.key(42)
  eps_b = jax.random.normal(
      jax.random.fold_in(nk, 0), (1, _F), dtype=jnp.float32).reshape(_B, 2)
  eps_e = jax.random.normal(
      jax.random.fold_in(nk, 1), (1, _F, _D), dtype=jnp.float32
  ).reshape(_B, 2 * _D)
  return eps_b, eps_e


def _sc_gather(idx2d, blk2d, bias_blocks, entity_table):
  mesh = plsc.VectorSubcoreMesh(core_axis_name="c", subcore_axis_name="s")

  @functools.partial(
      pl.kernel,
      mesh=mesh,
      compiler_params=pltpu.CompilerParams(use_tc_tiling_on_sc=False),
      out_type=[
          jax.ShapeDtypeStruct((_F, _BBLK), jnp.float32),
          jax.ShapeDtypeStruct((_F, 2 * _D), jnp.float32),
      ],
      scratch_types=[
          pltpu.VMEM((_NCHUNK, _CHUNK), jnp.int32),
          pltpu.VMEM((_NCHUNK, _CHUNK), jnp.int32),
          pltpu.VMEM((_PER_W, _BBLK), jnp.float32),
          pltpu.VMEM((_PER_W, 2 * _D), jnp.float32),
          pltpu.SemaphoreType.DMA,
          pltpu.SemaphoreType.DMA,
      ],
  )
  def gather(idx_hbm, blk_hbm, bias_hbm, ent_hbm, bias_out, ent_out,
             idx_v, blk_v, bias_v, ent_v, sem_b, sem_e):
    wid = lax.axis_index("s") * _NC + lax.axis_index("c")
    base = wid * _PER_W
    pltpu.sync_copy(idx_hbm.at[pl.ds(wid * _NCHUNK, _NCHUNK)], idx_v)
    pltpu.sync_copy(blk_hbm.at[pl.ds(wid * _NCHUNK, _NCHUNK)], blk_v)
    waits = []
    for j in range(_NCHUNK):
      waits.append(pltpu.async_copy(
          ent_hbm.at[idx_v.at[j]], ent_v.at[pl.ds(j * _CHUNK, _CHUNK)], sem_e))
      waits.append(pltpu.async_copy(
          bias_hbm.at[blk_v.at[j]], bias_v.at[pl.ds(j * _CHUNK, _CHUNK)],
          sem_b))
    for w in waits:
      w.wait()
    pltpu.sync_copy(ent_v, ent_out.at[pl.ds(base, _PER_W)])
    pltpu.sync_copy(bias_v, bias_out.at[pl.ds(base, _PER_W)])

  return gather(idx2d, blk2d, bias_blocks, entity_table)


def _softplus(v):
  return jnp.logaddexp(v, 0.0)


_RB = 512


def _tc_body(b32_ref, mod_ref, ent_ref, epsb_ref, epse_ref, alpha_ref,
             mean_ref, std_ref, klu_ref, klv_ref):
  b32 = b32_ref[...]
  mod = mod_ref[...]
  e = ent_ref[...]
  eb = epsb_ref[...]
  ee = epse_ref[...]

  lanes = lax.broadcasted_iota(jnp.int32, (_RB, 2 * _BBLK), 1)
  mu = mod[:, 0:1]
  mv = mod[:, 1:2]
  zero = jnp.zeros_like(b32)

  def pick(target):
    return jnp.sum(jnp.where(lanes == target, b32, zero), axis=1,
                   keepdims=True)

  loc_u = pick(2 * mu)
  raw_u = pick(2 * mu + 1)
  loc_v = pick(_BBLK + 2 * mv)
  raw_v = pick(_BBLK + 2 * mv + 1)

  sp_u = _softplus(raw_u)
  sp_v = _softplus(raw_v)
  bias_part = loc_u + loc_v + sp_u * eb[:, 0:1] + sp_v * eb[:, 1:2]

  s_u = e[:, 0:_D] + _softplus(e[:, _D:2 * _D]) * ee[:, 0:_D]
  s_v = e[:, 2 * _D:3 * _D] + _softplus(e[:, 3 * _D:]) * ee[:, _D:]
  emb = jnp.sum(s_u * s_v, axis=1, keepdims=True)

  mean_ref[...] = bias_part + emb
  klu_ref[...] = -jnp.log(sp_u) + (sp_u * sp_u + loc_u * loc_u) * 0.5 - 0.5
  klv_ref[...] = -jnp.log(sp_v) + (sp_v * sp_v + loc_v * loc_v) * 0.5 - 0.5

  @pl.when(pl.program_id(0) == 0)
  def _():
    std_ref[...] = jnp.sqrt(1.0 / _softplus(alpha_ref[...]))


def _tc_compute(bias32, mod2, ent128, epsb, epse, alpha11):
  grid = _B // _RB
  return pl.pallas_call(
      _tc_body,
      grid=(grid,),
      in_specs=[
          pl.BlockSpec((_RB, 2 * _BBLK), lambda i: (i, 0)),
          pl.BlockSpec((_RB, 2), lambda i: (i, 0)),
          pl.BlockSpec((_RB, 2 * 2 * _D), lambda i: (i, 0)),
          pl.BlockSpec((_RB, 2), lambda i: (i, 0)),
          pl.BlockSpec((_RB, 2 * _D), lambda i: (i, 0)),
          pl.BlockSpec((1, 1), lambda i: (0, 0)),
      ],
      out_specs=[
          pl.BlockSpec((_RB, 1), lambda i: (i, 0)),
          pl.BlockSpec((1, 1), lambda i: (0, 0)),
          pl.BlockSpec((_RB, 1), lambda i: (i, 0)),
          pl.BlockSpec((_RB, 1), lambda i: (i, 0)),
      ],
      out_shape=[
          jax.ShapeDtypeStruct((_B, 1), jnp.float32),
          jax.ShapeDtypeStruct((1, 1), jnp.float32),
          jax.ShapeDtypeStruct((_B, 1), jnp.float32),
          jax.ShapeDtypeStruct((_B, 1), jnp.float32),
      ],
  )(bias32, mod2, ent128, epsb, epse, alpha11)


def kernel(x, bias_table, entity_table, alpha):
  flat = x.reshape(_F)
  idx2d = flat.reshape(_F // _CHUNK, _CHUNK)
  blk2d = (flat >> 3).reshape(_F // _CHUNK, _CHUNK)
  mod2 = (flat & 7).reshape(_B, 2)
  bias_blocks = bias_table.reshape(-1, _BBLK)
  b16_g, ent_g = _sc_gather(idx2d, blk2d, bias_blocks, entity_table)
  bias32 = b16_g.reshape(_B, 2 * _BBLK)
  ent128 = ent_g.reshape(_B, 4 * _D)
  eps_b, eps_e = _eps()
  mean, std, klu, klv = _tc_compute(
      bias32, mod2, ent128, eps_b, eps_e, alpha.reshape(1, 1))
  kl = jnp.concatenate([klu, klv], axis=1).reshape(-1)
  return (mean.reshape(-1), std.reshape(-1), kl)
